# baseline (speedup 1.0000x reference)
import functools

import jax
import jax.numpy as jnp
from jax import lax
from jax.experimental import pallas as pl
from jax.experimental.pallas import tpu as pltpu
from jax.experimental.pallas import tpu_sc as plsc

_N = 262144
_C = 51
_CHUNK = 2048
_GROUPS = _CHUNK // 16


def _sc_body(fb_hbm, lbl_hbm, out_hbm, fb_v, lbl_v, out_v):
    info = plsc.get_sparse_core_info()
    nc = info.num_cores
    wid = lax.axis_index("s") * nc + lax.axis_index("c")
    nw = nc * info.num_subcores
    rows_per_w = _N // nw
    base = wid * rows_per_w
    iota = lax.iota(jnp.int32, 16)
    n_chunks = rows_per_w // _CHUNK

    def chunk_body(ci, _):
        row0 = pl.multiple_of(base + ci * _CHUNK, _CHUNK)
        pltpu.sync_copy(fb_hbm.at[pl.ds(row0, _CHUNK), :], fb_v)
        pltpu.sync_copy(lbl_hbm.at[pl.ds(row0, _CHUNK)], lbl_v)

        def group_body(g, _):
            r = pl.multiple_of(g * 16, 16)
            rows = r + iota
            m = plsc.load_gather(fb_v, [rows, jnp.full((16,), 0, jnp.int32)])
            mi = jnp.zeros((16,), jnp.int32)
            for j in range(1, _C):
                v = plsc.load_gather(fb_v, [rows, jnp.full((16,), j, jnp.int32)])
                pred = v > m
                m = jnp.maximum(m, v)
                mi = jnp.where(pred, jnp.full((16,), j, jnp.int32), mi)
            lbl = lbl_v[pl.ds(r, 16)]
            out_v[pl.ds(r, 16)] = jnp.where(lbl == 0, mi, lbl)
            return 0

        lax.fori_loop(0, _GROUPS, group_body, 0)
        pltpu.sync_copy(out_v, out_hbm.at[pl.ds(row0, _CHUNK)])
        return 0

    lax.fori_loop(0, n_chunks, chunk_body, 0)


def kernel(rel_logits, freq_bias, rel_labels, rel_covar, gamma):
    n, c = freq_bias.shape
    run = pl.kernel(
        _sc_body,
        out_type=jax.ShapeDtypeStruct((n,), jnp.int32),
        mesh=plsc.VectorSubcoreMesh(core_axis_name="c", subcore_axis_name="s"),
        scratch_types=[
            pltpu.VMEM((_CHUNK, _C), jnp.float32),
            pltpu.VMEM((_CHUNK,), jnp.int32),
            pltpu.VMEM((_CHUNK,), jnp.int32),
        ],
        compiler_params=pltpu.CompilerParams(
            needs_layout_passes=False, use_tc_tiling_on_sc=False
        ),
    )
    return run(freq_bias, rel_labels)


# speedup vs baseline: 1.0119x; 1.0119x over previous
import functools

import jax
import jax.numpy as jnp
from jax import lax
from jax.experimental import pallas as pl
from jax.experimental.pallas import tpu as pltpu
from jax.experimental.pallas import tpu_sc as plsc

_N = 262144
_C = 51
_CHUNK = 2048
_GROUPS = _CHUNK // 16


def _sc_body(fb_hbm, lbl_hbm, out_hbm, fb_v, lbl_v, out_v):
    info = plsc.get_sparse_core_info()
    nc = info.num_cores
    wid = lax.axis_index("s") * nc + lax.axis_index("c")
    nw = nc * info.num_subcores
    rows_per_w = _N // nw
    base = wid * rows_per_w
    iota = lax.iota(jnp.int32, 16)
    n_chunks = rows_per_w // _CHUNK

    def chunk_body(ci, _):
        row0 = pl.multiple_of(base + ci * _CHUNK, _CHUNK)
        pltpu.sync_copy(fb_hbm.at[pl.ds(row0, _CHUNK), :], fb_v)
        pltpu.sync_copy(lbl_hbm.at[pl.ds(row0, _CHUNK)], lbl_v)

        def group_body(g, _):
            r = pl.multiple_of(g * 64, 64)
            rows = [r + k * 16 + iota for k in range(4)]
            m = [plsc.load_gather(fb_v, [rows[k], jnp.full((16,), 0, jnp.int32)])
                 for k in range(4)]
            mi = [jnp.zeros((16,), jnp.int32) for _ in range(4)]
            for j in range(1, _C):
                jv = jnp.full((16,), j, jnp.int32)
                v = [plsc.load_gather(fb_v, [rows[k], jv]) for k in range(4)]
                for k in range(4):
                    pred = v[k] > m[k]
                    m[k] = jnp.maximum(m[k], v[k])
                    mi[k] = jnp.where(pred, jv, mi[k])
            for k in range(4):
                lbl = lbl_v[pl.ds(pl.multiple_of(r + k * 16, 16), 16)]
                out_v[pl.ds(pl.multiple_of(r + k * 16, 16), 16)] = jnp.where(
                    lbl == 0, mi[k], lbl)
            return 0

        lax.fori_loop(0, _GROUPS // 4, group_body, 0)
        pltpu.sync_copy(out_v, out_hbm.at[pl.ds(row0, _CHUNK)])
        return 0

    lax.fori_loop(0, n_chunks, chunk_body, 0)


def kernel(rel_logits, freq_bias, rel_labels, rel_covar, gamma):
    n, c = freq_bias.shape
    run = pl.kernel(
        _sc_body,
        out_type=jax.ShapeDtypeStruct((n,), jnp.int32),
        mesh=plsc.VectorSubcoreMesh(core_axis_name="c", subcore_axis_name="s"),
        scratch_types=[
            pltpu.VMEM((_CHUNK, _C), jnp.float32),
            pltpu.VMEM((_CHUNK,), jnp.int32),
            pltpu.VMEM((_CHUNK,), jnp.int32),
        ],
        compiler_params=pltpu.CompilerParams(
            needs_layout_passes=False, use_tc_tiling_on_sc=False
        ),
    )
    return run(freq_bias, rel_labels)


# speedup vs baseline: 1.0414x; 1.0292x over previous
import functools

import jax
import jax.numpy as jnp
from jax import lax
from jax.experimental import pallas as pl
from jax.experimental.pallas import tpu as pltpu
from jax.experimental.pallas import tpu_sc as plsc

_N = 262144
_C = 51
_CHUNK = 512
_GROUPS = _CHUNK // 16


def _sc_body(fb_hbm, lbl_hbm, out_hbm, fb_v, lbl_v, out_v):
    info = plsc.get_sparse_core_info()
    nc = info.num_cores
    wid = lax.axis_index("s") * nc + lax.axis_index("c")
    nw = nc * info.num_subcores
    rows_per_w = _N // nw
    base = wid * rows_per_w
    iota = lax.iota(jnp.int32, 16)
    n_chunks = rows_per_w // _CHUNK

    def chunk_body(ci, _):
        row0 = pl.multiple_of(base + ci * _CHUNK, _CHUNK)
        pltpu.sync_copy(fb_hbm.at[pl.ds(row0, _CHUNK), :], fb_v)
        pltpu.sync_copy(lbl_hbm.at[pl.ds(row0, _CHUNK)], lbl_v)

        def group_body(g, _):
            r = pl.multiple_of(g * 64, 64)
            rows = [r + k * 16 + iota for k in range(4)]
            m = [plsc.load_gather(fb_v, [rows[k], jnp.full((16,), 0, jnp.int32)])
                 for k in range(4)]
            mi = [jnp.zeros((16,), jnp.int32) for _ in range(4)]
            for j in range(1, _C):
                jv = jnp.full((16,), j, jnp.int32)
                v = [plsc.load_gather(fb_v, [rows[k], jv]) for k in range(4)]
                for k in range(4):
                    pred = v[k] > m[k]
                    m[k] = jnp.maximum(m[k], v[k])
                    mi[k] = jnp.where(pred, jv, mi[k])
            for k in range(4):
                lbl = lbl_v[pl.ds(pl.multiple_of(r + k * 16, 16), 16)]
                out_v[pl.ds(pl.multiple_of(r + k * 16, 16), 16)] = jnp.where(
                    lbl == 0, mi[k], lbl)
            return 0

        lax.fori_loop(0, _GROUPS // 4, group_body, 0)
        pltpu.sync_copy(out_v, out_hbm.at[pl.ds(row0, _CHUNK)])
        return 0

    lax.fori_loop(0, n_chunks, chunk_body, 0)


def kernel(rel_logits, freq_bias, rel_labels, rel_covar, gamma):
    n, c = freq_bias.shape
    run = pl.kernel(
        _sc_body,
        out_type=jax.ShapeDtypeStruct((n,), jnp.int32),
        mesh=plsc.VectorSubcoreMesh(core_axis_name="c", subcore_axis_name="s"),
        scratch_types=[
            pltpu.VMEM((_CHUNK, _C), jnp.float32),
            pltpu.VMEM((_CHUNK,), jnp.int32),
            pltpu.VMEM((_CHUNK,), jnp.int32),
        ],
        compiler_params=pltpu.CompilerParams(
            needs_layout_passes=False, use_tc_tiling_on_sc=True
        ),
    )
    return run(freq_bias, rel_labels)


# speedup vs baseline: 2.7589x; 2.6492x over previous
import jax
import jax.numpy as jnp
from jax.experimental import pallas as pl
from jax.experimental.pallas import tpu as pltpu


_BLOCK = 16384


def _rows_kernel(fb_ref, lbl_ref, out_ref):
    ft = fb_ref[...].T
    idx = jnp.argmax(ft, axis=0).astype(jnp.int32)
    lbl = lbl_ref[0, 0, :]
    out_ref[0, 0, :] = jnp.where(lbl == 0, idx, lbl)


def kernel(rel_logits, freq_bias, rel_labels, rel_covar, gamma):
    n, c = freq_bias.shape
    grid = n // _BLOCK
    lbl3 = rel_labels.reshape(grid, 1, _BLOCK)
    out = pl.pallas_call(
        _rows_kernel,
        grid=(grid,),
        in_specs=[
            pl.BlockSpec((_BLOCK, c), lambda i: (i, 0)),
            pl.BlockSpec((1, 1, _BLOCK), lambda i: (i, 0, 0)),
        ],
        out_specs=pl.BlockSpec((1, 1, _BLOCK), lambda i: (i, 0, 0)),
        out_shape=jax.ShapeDtypeStruct((grid, 1, _BLOCK), jnp.int32),
        compiler_params=pltpu.CompilerParams(
            dimension_semantics=("arbitrary",),
            allow_input_fusion=[True, True],
        ),
    )(freq_bias, lbl3)
    return out.reshape(n)
